# Initial kernel scaffold; baseline (speedup 1.0000x reference)
import jax
import jax.numpy as jnp
from jax.experimental import pallas as pl


def kernel(X, table):
    raise NotImplementedError("write your pallas kernel here")


# speedup vs baseline: 3.3372x; 3.3372x over previous
import functools

import jax
import jax.numpy as jnp
from jax import lax
from jax.experimental import pallas as pl
from jax.experimental.pallas import tpu as pltpu
from jax.experimental.pallas import tpu_sc as plsc

_INFO = plsc.get_sparse_core_info()
_NC = _INFO.num_cores
_NS = _INFO.num_subcores
_NW = _NC * _NS


def _make_gather(B, V, D, chunk):
    assert B % _NW == 0
    b_per_w = B // _NW
    assert b_per_w % chunk == 0
    n_chunks = b_per_w // chunk

    mesh = plsc.VectorSubcoreMesh(core_axis_name="c", subcore_axis_name="s")

    @functools.partial(
        pl.kernel,
        mesh=mesh,
        out_type=jax.ShapeDtypeStruct((B, D), jnp.float32),
        scratch_types=[
            pltpu.VMEM((b_per_w,), jnp.int32),
            pltpu.VMEM((chunk, D), jnp.float32),
            pltpu.VMEM((chunk, D), jnp.float32),
            pltpu.SemaphoreType.DMA,
            pltpu.SemaphoreType.DMA,
            pltpu.SemaphoreType.DMA,
        ],
    )
    def gather_kernel(idx_hbm, table_hbm, out_hbm, idx_v, rows0, rows1,
                      sem_idx, sem0, sem1):
        wid = lax.axis_index("s") * _NC + lax.axis_index("c")
        base = wid * b_per_w

        pltpu.async_copy(idx_hbm.at[pl.ds(base, b_per_w)], idx_v,
                         sem_idx).wait()

        rows = (rows0, rows1)
        sems = (sem0, sem1)

        def fire(k, buf):
            return pltpu.async_copy(
                table_hbm.at[idx_v.at[pl.ds(k * chunk, chunk)]],
                rows[buf], sems[buf])

        def drain(k, buf):
            pltpu.sync_copy(rows[buf],
                            out_hbm.at[pl.ds(base + k * chunk, chunk)])

        cp = fire(0, 0)
        for k in range(n_chunks):
            buf = k % 2
            if k + 1 < n_chunks:
                nxt = fire(k + 1, 1 - buf)
            cp.wait()
            drain(k, buf)
            if k + 1 < n_chunks:
                cp = nxt

    return gather_kernel


def kernel(X, table):
    Bdim, H = X.shape
    V, D = table.shape
    B = Bdim * H
    idx = X.reshape(B).astype(jnp.int32)
    out = _make_gather(B, V, D, chunk=400)(idx, table)
    return out.reshape(Bdim, H, D)
